# Initial kernel scaffold; baseline (speedup 1.0000x reference)
import jax
import jax.numpy as jnp
from jax import lax
from jax.experimental import pallas as pl
from jax.experimental.pallas import tpu as pltpu
from jax.experimental.pallas import tpu_sc as plsc

F32 = jnp.float32
N = 50000
E = 800000
NF = 128
H = 64
G = 128
HH = H // 2

CH = 128
NCHUNK = 6400
E_PAD = NCHUNK * CH
PAD = E_PAD - E
ACC_ROWS = N + 8

NS = 16
NC = 2
TILE_ROWS = N // NS
ZROWS = 625

BN = 2000
NB = N // BN

_MESH = plsc.VectorSubcoreMesh(
    core_axis_name="c", subcore_axis_name="s", num_cores=NC, num_subcores=NS)


def _deg_body(dst_hbm, ones_hbm, zeros8_hbm, outa_hbm, outb_hbm,
              dstbuf, ones_v, zbuf8, deg_sh):
    c = lax.axis_index("c")
    s = lax.axis_index("s")
    pltpu.sync_copy(ones_hbm, ones_v)
    pltpu.sync_copy(zeros8_hbm, zbuf8)
    base = s * TILE_ROWS
    for r in range(5):
        pltpu.sync_copy(zbuf8, deg_sh.at[pl.ds(base + r * ZROWS, ZROWS)])

    @pl.when(s == 0)
    def _():
        pltpu.sync_copy(zbuf8.at[pl.ds(0, 8)], deg_sh.at[pl.ds(N, 8)])

    plsc.subcore_barrier()

    per_w = NCHUNK // (NC * NS)
    row0 = (c * NS + s) * per_w
    pltpu.sync_copy(dst_hbm.at[pl.ds(row0, per_w)], dstbuf)

    def step(j, carry):
        pltpu.sync_copy(ones_v, deg_sh.at[dstbuf.at[j]], add=True)
        return carry

    lax.fori_loop(0, per_w, step, 0)
    plsc.subcore_barrier()

    @pl.when(c == 0)
    def _():
        pltpu.sync_copy(deg_sh.at[pl.ds(base, TILE_ROWS)],
                        outa_hbm.at[pl.ds(base, TILE_ROWS)])

    @pl.when(c == 1)
    def _():
        pltpu.sync_copy(deg_sh.at[pl.ds(base, TILE_ROWS)],
                        outb_hbm.at[pl.ds(base, TILE_ROWS)])


def _conv_body(src_hbm, dst_hbm, hna_hbm, hnb_hbm, zeros_hbm,
               outa_hbm, outb_hbm,
               srcbuf, dstbuf, rows0, rows1, zbuf, acc_sh, sem0, sem1):
    c = lax.axis_index("c")
    s = lax.axis_index("s")
    pltpu.sync_copy(zeros_hbm, zbuf)
    base = s * TILE_ROWS
    for r in range(5):
        pltpu.sync_copy(zbuf, acc_sh.at[pl.ds(base + r * ZROWS, ZROWS)])

    @pl.when(s == 0)
    def _():
        pltpu.sync_copy(zbuf.at[pl.ds(0, 8)], acc_sh.at[pl.ds(N, 8)])

    plsc.subcore_barrier()

    per_tile = NCHUNK // NS
    half_ch = per_tile // 2

    def run(hn_hbm):
        bufs = ((rows0, sem0), (rows1, sem1))
        for half in range(2):
            row0 = s * per_tile + half * half_ch
            pltpu.sync_copy(src_hbm.at[pl.ds(row0, half_ch)], srcbuf)
            pltpu.sync_copy(dst_hbm.at[pl.ds(row0, half_ch)], dstbuf)
            for b2, (buf, sem) in enumerate(bufs):
                pltpu.async_copy(hn_hbm.at[srcbuf.at[b2]], buf, sem)

            def step(i, carry):
                j2 = i * 2
                for b2, (buf, sem) in enumerate(bufs):
                    j = j2 + b2
                    pltpu.make_async_copy(
                        hn_hbm.at[srcbuf.at[j]], buf, sem).wait()
                    pltpu.sync_copy(buf, acc_sh.at[dstbuf.at[j]], add=True)
                    pltpu.async_copy(hn_hbm.at[srcbuf.at[j + 2]], buf, sem)
                return carry

            lax.fori_loop(0, half_ch // 2 - 1, step, 0)
            for b2, (buf, sem) in enumerate(bufs):
                j = half_ch - 2 + b2
                pltpu.make_async_copy(hn_hbm.at[srcbuf.at[j]], buf, sem).wait()
                pltpu.sync_copy(buf, acc_sh.at[dstbuf.at[j]], add=True)

    @pl.when(c == 0)
    def _():
        run(hna_hbm)

    @pl.when(c == 1)
    def _():
        run(hnb_hbm)

    plsc.subcore_barrier()

    @pl.when(c == 0)
    def _():
        pltpu.sync_copy(acc_sh.at[pl.ds(base, TILE_ROWS)],
                        outa_hbm.at[pl.ds(base, TILE_ROWS)])

    @pl.when(c == 1)
    def _():
        pltpu.sync_copy(acc_sh.at[pl.ds(base, TILE_ROWS)],
                        outb_hbm.at[pl.ds(base, TILE_ROWS)])


_deg_call = pl.kernel(
    _deg_body,
    out_type=[jax.ShapeDtypeStruct((N, 8), F32)] * 2,
    mesh=_MESH,
    scratch_types=[
        pltpu.VMEM((NCHUNK // (NC * NS), CH), jnp.int32),
        pltpu.VMEM((CH, 8), F32),
        pltpu.VMEM((ZROWS, 8), F32),
        pltpu.VMEM_SHARED((ACC_ROWS, 8), F32),
    ],
)

_conv_call = pl.kernel(
    _conv_body,
    out_type=[jax.ShapeDtypeStruct((N, HH), F32)] * 2,
    mesh=_MESH,
    scratch_types=[
        pltpu.VMEM((NCHUNK // NS // 2, CH), jnp.int32),
        pltpu.VMEM((NCHUNK // NS // 2, CH), jnp.int32),
        pltpu.VMEM((CH, HH), F32),
        pltpu.VMEM((CH, HH), F32),
        pltpu.VMEM((ZROWS, HH), F32),
        pltpu.VMEM_SHARED((ACC_ROWS, HH), F32),
        pltpu.SemaphoreType.DMA,
        pltpu.SemaphoreType.DMA,
    ],
)


def _emb_body(x_ref, dega_ref, degb_ref, wemb_ref, bemb_ref, wc0_ref,
              hna_ref, hnb_ref, dis_ref):
    deg = dega_ref[:, :1] + degb_ref[:, :1] + 1.0
    dis = lax.rsqrt(deg)
    h0 = jnp.maximum(
        jnp.dot(x_ref[...], wemb_ref[...], preferred_element_type=F32)
        + bemb_ref[...], 0.0)
    hn = dis * jnp.dot(h0, wc0_ref[...], preferred_element_type=F32)
    hna_ref[...] = hn[:, :HH]
    hnb_ref[...] = hn[:, HH:]
    dis_ref[...] = dis


def _mid_body(acca_ref, accb_ref, hna_ref, hnb_ref, dis_ref, b_ref, w_ref,
              outa_ref, outb_ref):
    acc = jnp.concatenate([acca_ref[...], accb_ref[...]], axis=1)
    hn = jnp.concatenate([hna_ref[...], hnb_ref[...]], axis=1)
    dis = dis_ref[...]
    h = jnp.maximum(dis * (acc + hn) + b_ref[...], 0.0)
    hn2 = dis * jnp.dot(h, w_ref[...], preferred_element_type=F32)
    outa_ref[...] = hn2[:, :HH]
    outb_ref[...] = hn2[:, HH:]


def _pool_body(acca_ref, accb_ref, hna_ref, hnb_ref, dis_ref, b_ref,
               batch_ref, sums_ref, cnt_ref):
    i = pl.program_id(0)
    acc = jnp.concatenate([acca_ref[...], accb_ref[...]], axis=1)
    hn = jnp.concatenate([hna_ref[...], hnb_ref[...]], axis=1)
    h = jnp.maximum(dis_ref[...] * (acc + hn) + b_ref[...], 0.0)
    gids = lax.broadcasted_iota(jnp.int32, (G, 1), 0)
    onehot = jnp.where(batch_ref[0] == gids, 1.0, 0.0)
    psum = jnp.dot(onehot, h, preferred_element_type=F32)
    pcnt = jnp.sum(onehot, axis=1, keepdims=True)

    @pl.when(i == 0)
    def _():
        sums_ref[...] = jnp.zeros_like(sums_ref)
        cnt_ref[...] = jnp.zeros_like(cnt_ref)

    sums_ref[...] += psum
    cnt_ref[...] += pcnt


def _head_body(sums_ref, cnt_ref, w0_ref, b0_ref, w1_ref, b1_ref,
               w2_ref, b2_ref, out_ref):
    g = sums_ref[...] / jnp.maximum(cnt_ref[...], 1.0)
    h = jnp.maximum(
        jnp.dot(g, w0_ref[...], preferred_element_type=F32) + b0_ref[...], 0.0)
    h = jnp.maximum(
        jnp.dot(h, w1_ref[...], preferred_element_type=F32) + b1_ref[...], 0.0)
    out_ref[...] = (
        jnp.dot(h, w2_ref[...], preferred_element_type=F32) + b2_ref[...])


def _full(shape):
    nd = len(shape)
    return pl.BlockSpec(shape, lambda *_, __nd=nd: (0,) * __nd)


def _emb_tc(x, dega, degb, wemb, bemb, wc0):
    return pl.pallas_call(
        _emb_body,
        grid=(NB,),
        in_specs=[
            pl.BlockSpec((BN, NF), lambda i: (i, 0)),
            pl.BlockSpec((BN, 8), lambda i: (i, 0)),
            pl.BlockSpec((BN, 8), lambda i: (i, 0)),
            _full((NF, H)),
            _full((1, H)),
            _full((H, H)),
        ],
        out_specs=[
            pl.BlockSpec((BN, HH), lambda i: (i, 0)),
            pl.BlockSpec((BN, HH), lambda i: (i, 0)),
            pl.BlockSpec((BN, 1), lambda i: (i, 0)),
        ],
        out_shape=[
            jax.ShapeDtypeStruct((N, HH), F32),
            jax.ShapeDtypeStruct((N, HH), F32),
            jax.ShapeDtypeStruct((N, 1), F32),
        ],
    )(x, dega, degb, wemb, bemb, wc0)


def _mid_tc(acca, accb, hna, hnb, dis, b, w):
    return pl.pallas_call(
        _mid_body,
        grid=(NB,),
        in_specs=[
            pl.BlockSpec((BN, HH), lambda i: (i, 0)),
            pl.BlockSpec((BN, HH), lambda i: (i, 0)),
            pl.BlockSpec((BN, HH), lambda i: (i, 0)),
            pl.BlockSpec((BN, HH), lambda i: (i, 0)),
            pl.BlockSpec((BN, 1), lambda i: (i, 0)),
            _full((1, H)),
            _full((H, H)),
        ],
        out_specs=[
            pl.BlockSpec((BN, HH), lambda i: (i, 0)),
            pl.BlockSpec((BN, HH), lambda i: (i, 0)),
        ],
        out_shape=[
            jax.ShapeDtypeStruct((N, HH), F32),
            jax.ShapeDtypeStruct((N, HH), F32),
        ],
    )(acca, accb, hna, hnb, dis, b, w)


def _pool_tc(acca, accb, hna, hnb, dis, b, batch_r):
    return pl.pallas_call(
        _pool_body,
        grid=(NB,),
        in_specs=[
            pl.BlockSpec((BN, HH), lambda i: (i, 0)),
            pl.BlockSpec((BN, HH), lambda i: (i, 0)),
            pl.BlockSpec((BN, HH), lambda i: (i, 0)),
            pl.BlockSpec((BN, HH), lambda i: (i, 0)),
            pl.BlockSpec((BN, 1), lambda i: (i, 0)),
            _full((1, H)),
            pl.BlockSpec((1, 1, BN), lambda i: (i, 0, 0)),
        ],
        out_specs=[
            _full((G, H)),
            _full((G, 1)),
        ],
        out_shape=[
            jax.ShapeDtypeStruct((G, H), F32),
            jax.ShapeDtypeStruct((G, 1), F32),
        ],
    )(acca, accb, hna, hnb, dis, b, batch_r)


def _head_tc(sums, cnt, w0, b0, w1, b1, w2, b2):
    return pl.pallas_call(
        _head_body,
        grid=(1,),
        in_specs=[
            _full((G, H)),
            _full((G, 1)),
            _full((H, H)),
            _full((1, H)),
            _full((H, HH)),
            _full((1, HH)),
            _full((HH, 1)),
            _full((1, 1)),
        ],
        out_specs=_full((G, 1)),
        out_shape=jax.ShapeDtypeStruct((G, 1), F32),
    )(sums, cnt, w0, b0, w1, b1, w2, b2)


def kernel(x, edge_index, batch, W_emb, b_emb, Wc0, bc0, Wc1, bc1, Wc2, bc2,
           Wr0, br0, Wr1, br1, Wr2, br2):
    padi = jnp.arange(PAD, dtype=jnp.int32)
    src_r = jnp.concatenate([edge_index[0], (padi * 37) % N]).reshape(NCHUNK, CH)
    dst_r = jnp.concatenate([edge_index[1], N + (padi % 8)]).reshape(NCHUNK, CH)
    zeros32 = jnp.zeros((ZROWS, HH), F32)
    zeros8 = jnp.zeros((ZROWS, 8), F32)
    ones8 = jnp.ones((CH, 8), F32)
    batch_r = batch.reshape(NB, 1, BN)

    dega, degb = _deg_call(dst_r, ones8, zeros8)
    hna, hnb, dis = _emb_tc(x, dega, degb, W_emb, b_emb.reshape(1, H), Wc0)
    acca, accb = _conv_call(src_r, dst_r, hna, hnb, zeros32)
    hna, hnb = _mid_tc(acca, accb, hna, hnb, dis, bc0.reshape(1, H), Wc1)
    acca, accb = _conv_call(src_r, dst_r, hna, hnb, zeros32)
    hna, hnb = _mid_tc(acca, accb, hna, hnb, dis, bc1.reshape(1, H), Wc2)
    acca, accb = _conv_call(src_r, dst_r, hna, hnb, zeros32)
    sums, cnt = _pool_tc(acca, accb, hna, hnb, dis, bc2.reshape(1, H), batch_r)
    return _head_tc(sums, cnt, Wr0, br0.reshape(1, H), Wr1, br1.reshape(1, HH),
                    Wr2, br2.reshape(1, 1))


# speedup vs baseline: 22.9932x; 22.9932x over previous
import jax
import jax.numpy as jnp
from jax import lax
from jax.experimental import pallas as pl
from jax.experimental.pallas import tpu as pltpu
from jax.experimental.pallas import tpu_sc as plsc

F32 = jnp.float32
N = 50000
E = 800000
NF = 128
H = 64
G = 128
HH = H // 2

CH = 128
NCHUNK = 6400
E_PAD = NCHUNK * CH
PAD = E_PAD - E
NPAD = 50048
ACC_ROWS = NPAD

NS = 16
NC = 2
TILE_ROWS = NPAD // NS
ZROWS = 184
NZ = TILE_ROWS // ZROWS

ROUND_CH = 50

BN = 2000
NB = N // BN

_MESH = plsc.VectorSubcoreMesh(
    core_axis_name="c", subcore_axis_name="s", num_cores=NC, num_subcores=NS)


def _deg_body(dst_hbm, ones_hbm, zeros8_hbm, out_hbm,
              dstbuf, ones_v, zbuf8, deg_sh):
    c = lax.axis_index("c")
    s = lax.axis_index("s")
    pltpu.sync_copy(ones_hbm, ones_v)
    pltpu.sync_copy(zeros8_hbm, zbuf8)
    base = s * TILE_ROWS
    for r in range(NZ):
        pltpu.sync_copy(zbuf8, deg_sh.at[pl.ds(base + r * ZROWS, ZROWS)])
    plsc.subcore_barrier()

    per_w = NCHUNK // (NC * NS)
    row0 = (c * NS + s) * per_w
    pltpu.sync_copy(dst_hbm.at[pl.ds(row0, per_w)], dstbuf)

    def step(j, carry):
        pltpu.sync_copy(ones_v, deg_sh.at[dstbuf.at[j]], add=True)
        return carry

    lax.fori_loop(0, per_w, step, 0)
    plsc.subcore_barrier()
    pltpu.sync_copy(deg_sh.at[pl.ds(base, TILE_ROWS)],
                    out_hbm.at[pl.ds(c * NPAD + base, TILE_ROWS)])


def _conv_body(src_hbm, dst_hbm, hna_hbm, hnb_hbm, zeros_hbm,
               outa_hbm, outb_hbm,
               srcbuf, dstbuf, rows0, rows1, zbuf, acc_sh, sem0, sem1):
    c = lax.axis_index("c")
    s = lax.axis_index("s")
    pltpu.sync_copy(zeros_hbm, zbuf)
    base = s * TILE_ROWS
    for r in range(NZ):
        pltpu.sync_copy(zbuf, acc_sh.at[pl.ds(base + r * ZROWS, ZROWS)])
    plsc.subcore_barrier()

    per_tile = NCHUNK // NS

    def run(hn_hbm):
        bufs = ((rows0, sem0), (rows1, sem1))

        def round_body(rnd, carry):
            row0 = s * per_tile + rnd * ROUND_CH
            pltpu.sync_copy(src_hbm.at[pl.ds(row0, ROUND_CH)], srcbuf)
            pltpu.sync_copy(dst_hbm.at[pl.ds(row0, ROUND_CH)], dstbuf)
            for b2, (buf, sem) in enumerate(bufs):
                pltpu.async_copy(hn_hbm.at[srcbuf.at[b2]], buf, sem)

            def step(i, c2):
                j2 = i * 2
                for b2, (buf, sem) in enumerate(bufs):
                    j = j2 + b2
                    pltpu.make_async_copy(
                        hn_hbm.at[srcbuf.at[j]], buf, sem).wait()
                    pltpu.sync_copy(buf, acc_sh.at[dstbuf.at[j]], add=True)
                    pltpu.async_copy(hn_hbm.at[srcbuf.at[j + 2]], buf, sem)
                return c2

            lax.fori_loop(0, ROUND_CH // 2 - 1, step, 0)
            for b2, (buf, sem) in enumerate(bufs):
                j = ROUND_CH - 2 + b2
                pltpu.make_async_copy(hn_hbm.at[srcbuf.at[j]], buf, sem).wait()
                pltpu.sync_copy(buf, acc_sh.at[dstbuf.at[j]], add=True)
            return carry

        lax.fori_loop(0, per_tile // ROUND_CH, round_body, 0)

    @pl.when(c == 0)
    def _():
        run(hna_hbm)

    @pl.when(c == 1)
    def _():
        run(hnb_hbm)

    plsc.subcore_barrier()

    @pl.when(c == 0)
    def _():
        pltpu.sync_copy(acc_sh.at[pl.ds(base, TILE_ROWS)],
                        outa_hbm.at[pl.ds(base, TILE_ROWS)])

    @pl.when(c == 1)
    def _():
        pltpu.sync_copy(acc_sh.at[pl.ds(base, TILE_ROWS)],
                        outb_hbm.at[pl.ds(base, TILE_ROWS)])


_deg_call = pl.kernel(
    _deg_body,
    out_type=jax.ShapeDtypeStruct((2 * NPAD, 8), F32),
    mesh=_MESH,
    scratch_types=[
        pltpu.VMEM((NCHUNK // (NC * NS), CH), jnp.int32),
        pltpu.VMEM((CH, 8), F32),
        pltpu.VMEM((ZROWS, 8), F32),
        pltpu.VMEM_SHARED((ACC_ROWS, 8), F32),
    ],
    compiler_params=pltpu.CompilerParams(use_tc_tiling_on_sc=False),
)

_conv_call = pl.kernel(
    _conv_body,
    out_type=[jax.ShapeDtypeStruct((NPAD, HH), F32)] * 2,
    mesh=_MESH,
    scratch_types=[
        pltpu.VMEM((ROUND_CH, CH), jnp.int32),
        pltpu.VMEM((ROUND_CH, CH), jnp.int32),
        pltpu.VMEM((CH, HH), F32),
        pltpu.VMEM((CH, HH), F32),
        pltpu.VMEM((ZROWS, HH), F32),
        pltpu.VMEM_SHARED((ACC_ROWS, HH), F32),
        pltpu.SemaphoreType.DMA,
        pltpu.SemaphoreType.DMA,
    ],
    compiler_params=pltpu.CompilerParams(use_tc_tiling_on_sc=False),
)


def _emb_body(x_ref, dis_ref, wemb_ref, bemb_ref, wc0_ref,
              hna_ref, hnb_ref):
    dis = dis_ref[...]
    h0 = jnp.maximum(
        jnp.dot(x_ref[...], wemb_ref[...], preferred_element_type=F32, precision=lax.Precision.HIGHEST)
        + bemb_ref[...], 0.0)
    hn = dis * jnp.dot(h0, wc0_ref[...], preferred_element_type=F32, precision=lax.Precision.HIGHEST)
    hna_ref[...] = hn[:, :HH]
    hnb_ref[...] = hn[:, HH:]


def _mid_body(acca_ref, accb_ref, hna_ref, hnb_ref, dis_ref, b_ref, w_ref,
              outa_ref, outb_ref):
    acc = jnp.concatenate([acca_ref[...], accb_ref[...]], axis=1)
    hn = jnp.concatenate([hna_ref[...], hnb_ref[...]], axis=1)
    dis = dis_ref[...]
    h = jnp.maximum(dis * (acc + hn) + b_ref[...], 0.0)
    hn2 = dis * jnp.dot(h, w_ref[...], preferred_element_type=F32, precision=lax.Precision.HIGHEST)
    outa_ref[...] = hn2[:, :HH]
    outb_ref[...] = hn2[:, HH:]


def _pool_body(acca_ref, accb_ref, hna_ref, hnb_ref, dis_ref, b_ref,
               batch_ref, sums_ref, cnt_ref):
    i = pl.program_id(0)
    acc = jnp.concatenate([acca_ref[...], accb_ref[...]], axis=1)
    hn = jnp.concatenate([hna_ref[...], hnb_ref[...]], axis=1)
    h = jnp.maximum(dis_ref[...] * (acc + hn) + b_ref[...], 0.0)
    gids = lax.broadcasted_iota(jnp.int32, (G, 1), 0)
    onehot = jnp.where(batch_ref[0] == gids, 1.0, 0.0)
    psum = jnp.dot(onehot, h, preferred_element_type=F32, precision=lax.Precision.HIGHEST)
    pcnt = jnp.sum(onehot, axis=1, keepdims=True)

    @pl.when(i == 0)
    def _():
        sums_ref[...] = jnp.zeros_like(sums_ref)
        cnt_ref[...] = jnp.zeros_like(cnt_ref)

    sums_ref[...] += psum
    cnt_ref[...] += pcnt


def _head_body(sums_ref, cnt_ref, w0_ref, b0_ref, w1_ref, b1_ref,
               w2_ref, b2_ref, out_ref):
    g = sums_ref[...] / jnp.maximum(cnt_ref[...], 1.0)
    h = jnp.maximum(
        jnp.dot(g, w0_ref[...], preferred_element_type=F32, precision=lax.Precision.HIGHEST) + b0_ref[...], 0.0)
    h = jnp.maximum(
        jnp.dot(h, w1_ref[...], preferred_element_type=F32, precision=lax.Precision.HIGHEST) + b1_ref[...], 0.0)
    out_ref[...] = (
        jnp.dot(h, w2_ref[...], preferred_element_type=F32, precision=lax.Precision.HIGHEST) + b2_ref[...])


def _full(shape):
    nd = len(shape)
    return pl.BlockSpec(shape, lambda *_, __nd=nd: (0,) * __nd)


def _emb_tc(x, dis, wemb, bemb, wc0):
    return pl.pallas_call(
        _emb_body,
        grid=(NB,),
        in_specs=[
            pl.BlockSpec((BN, NF), lambda i: (i, 0)),
            pl.BlockSpec((BN, 1), lambda i: (i, 0)),
            _full((NF, H)),
            _full((1, H)),
            _full((H, H)),
        ],
        out_specs=[
            pl.BlockSpec((BN, HH), lambda i: (i, 0)),
            pl.BlockSpec((BN, HH), lambda i: (i, 0)),
        ],
        out_shape=[
            jax.ShapeDtypeStruct((N, HH), F32),
            jax.ShapeDtypeStruct((N, HH), F32),
        ],
    )(x, dis, wemb, bemb, wc0)


def _mid_tc(acca, accb, hna, hnb, dis, b, w):
    return pl.pallas_call(
        _mid_body,
        grid=(NB,),
        in_specs=[
            pl.BlockSpec((BN, HH), lambda i: (i, 0)),
            pl.BlockSpec((BN, HH), lambda i: (i, 0)),
            pl.BlockSpec((BN, HH), lambda i: (i, 0)),
            pl.BlockSpec((BN, HH), lambda i: (i, 0)),
            pl.BlockSpec((BN, 1), lambda i: (i, 0)),
            _full((1, H)),
            _full((H, H)),
        ],
        out_specs=[
            pl.BlockSpec((BN, HH), lambda i: (i, 0)),
            pl.BlockSpec((BN, HH), lambda i: (i, 0)),
        ],
        out_shape=[
            jax.ShapeDtypeStruct((N, HH), F32),
            jax.ShapeDtypeStruct((N, HH), F32),
        ],
    )(acca, accb, hna, hnb, dis, b, w)


def _pool_tc(acca, accb, hna, hnb, dis, b, batch_r):
    return pl.pallas_call(
        _pool_body,
        grid=(NB,),
        in_specs=[
            pl.BlockSpec((BN, HH), lambda i: (i, 0)),
            pl.BlockSpec((BN, HH), lambda i: (i, 0)),
            pl.BlockSpec((BN, HH), lambda i: (i, 0)),
            pl.BlockSpec((BN, HH), lambda i: (i, 0)),
            pl.BlockSpec((BN, 1), lambda i: (i, 0)),
            _full((1, H)),
            pl.BlockSpec((1, 1, BN), lambda i: (i, 0, 0)),
        ],
        out_specs=[
            _full((G, H)),
            _full((G, 1)),
        ],
        out_shape=[
            jax.ShapeDtypeStruct((G, H), F32),
            jax.ShapeDtypeStruct((G, 1), F32),
        ],
    )(acca, accb, hna, hnb, dis, b, batch_r)


def _head_tc(sums, cnt, w0, b0, w1, b1, w2, b2):
    return pl.pallas_call(
        _head_body,
        grid=(1,),
        in_specs=[
            _full((G, H)),
            _full((G, 1)),
            _full((H, H)),
            _full((1, H)),
            _full((H, HH)),
            _full((1, HH)),
            _full((HH, 1)),
            _full((1, 1)),
        ],
        out_specs=_full((G, 1)),
        out_shape=jax.ShapeDtypeStruct((G, 1), F32),
    )(sums, cnt, w0, b0, w1, b1, w2, b2)


def kernel(x, edge_index, batch, W_emb, b_emb, Wc0, bc0, Wc1, bc1, Wc2, bc2,
           Wr0, br0, Wr1, br1, Wr2, br2):
    padi = jnp.arange(PAD, dtype=jnp.int32)
    src_r = jnp.concatenate([edge_index[0], (padi * 37) % N]).reshape(NCHUNK, CH)
    dst_r = jnp.concatenate([edge_index[1], N + (padi % (NPAD - N))]).reshape(NCHUNK, CH)
    zeros32 = jnp.zeros((ZROWS, HH), F32)
    zeros8 = jnp.zeros((ZROWS, 8), F32)
    ones8 = jnp.ones((CH, 8), F32)
    batch_r = batch.reshape(NB, 1, BN)

    degcat = _deg_call(dst_r, ones8, zeros8)
    deg = degcat[:N, 0] + degcat[NPAD:NPAD + N, 0] + 1.0
    dis = jnp.where(deg > 0, deg ** -0.5, 0.0).reshape(N, 1)
    hna, hnb = _emb_tc(x, dis, W_emb, b_emb.reshape(1, H), Wc0)
    acca, accb = _conv_call(src_r, dst_r, hna, hnb, zeros32)
    hna, hnb = _mid_tc(acca, accb, hna, hnb, dis, bc0.reshape(1, H), Wc1)
    acca, accb = _conv_call(src_r, dst_r, hna, hnb, zeros32)
    hna, hnb = _mid_tc(acca, accb, hna, hnb, dis, bc1.reshape(1, H), Wc2)
    acca, accb = _conv_call(src_r, dst_r, hna, hnb, zeros32)
    sums, cnt = _pool_tc(acca, accb, hna, hnb, dis, bc2.reshape(1, H), batch_r)
    return _head_tc(sums, cnt, Wr0, br0.reshape(1, H), Wr1, br1.reshape(1, HH),
                    Wr2, br2.reshape(1, 1))
